# baseline (speedup 1.0000x reference)
import jax
import jax.numpy as jnp
from jax.experimental import pallas as pl

H = 256
W = 256
N = H * W
S = 8
HC = H // S
B = 4
NIT = 30


def _shift_flat(x, dh, dw, col_ids):
    s = W * dh + dw
    C = x.shape[0]
    if s > 0:
        x = jnp.concatenate([jnp.zeros((C, s), jnp.float32), x[:, : N - s]], axis=1)
    elif s < 0:
        x = jnp.concatenate([x[:, -s:], jnp.zeros((C, -s), jnp.float32)], axis=1)
    if dw == 1:
        x = jnp.where(col_ids == 0, 0.0, x)
    elif dw == -1:
        x = jnp.where(col_ids == W - 1, 0.0, x)
    return x


def _conv_kernel(g_ref, yb_ref, w1_ref, b1_ref, w2_ref, b2_ref, w3_ref, b3_ref,
                 mu_ref, var_ref, aff_ref):
    col_ids = jax.lax.broadcasted_iota(jnp.int32, (1, N), 1) % W
    x0 = jnp.concatenate([g_ref[0], yb_ref[0]], axis=0)

    cols = [_shift_flat(x0, 1 - i, 1 - j, col_ids) for i in range(3) for j in range(3)]
    im2col = jnp.concatenate(cols, axis=0)
    l1 = jax.lax.dot(w1_ref[...], im2col, preferred_element_type=jnp.float32)
    l1 = l1 + b1_ref[...]
    var_ref[0] = l1[32:33]
    f = jnp.maximum(l1[:32], 0.0)

    CH = 16384
    PAD = 512
    for w_ref, b_ref, relu in ((w2_ref, b2_ref, True), (w3_ref, b3_ref, False)):
        fpad = jnp.concatenate(
            [jnp.zeros((32, PAD), jnp.float32), f,
             jnp.zeros((32, PAD), jnp.float32)], axis=1)
        outs = []
        for c in range(N // CH):
            base = c * CH
            cols = []
            for i in range(3):
                for j in range(3):
                    s = W * (1 - i) + (1 - j)
                    st = base + PAD - s
                    sl = fpad[:, st:st + CH]
                    cc = col_ids[:, :CH]
                    if j == 0:
                        sl = jnp.where(cc == 0, 0.0, sl)
                    elif j == 2:
                        sl = jnp.where(cc == W - 1, 0.0, sl)
                    cols.append(sl)
            im2col_c = jnp.concatenate(cols, axis=0)
            outs.append(jax.lax.dot(w_ref[...], im2col_c,
                                    preferred_element_type=jnp.float32))
        f = jnp.concatenate(outs, axis=1) + b_ref[...]
        if relu:
            f = jnp.maximum(f, 0.0)

    mu = mu_ref[0, 0]
    row_ids = jax.lax.broadcasted_iota(jnp.int32, (1, N), 1) // W

    def aff(dh, dw, border_ids, border_val):
        fn = _shift_flat(f, dh, dw, col_ids)
        d2 = jnp.sum((f - fn) ** 2, axis=0, keepdims=True)
        wdir = jnp.exp(-d2 / mu)
        return jnp.where(border_ids == border_val, 0.0, wdir)

    wu = aff(1, 0, row_ids, 0)
    wd = aff(-1, 0, row_ids, H - 1)
    wl = aff(0, 1, col_ids, 0)
    wr = aff(0, -1, col_ids, W - 1)
    deg = wu + wd + wl + wr
    aff_ref[0] = jnp.concatenate([wu, wd, wl, wr, deg], axis=0)


def _cg_kernel(wu_ref, wd_ref, wl_ref, wr_ref, deg_ref, src_ref, mask_ref,
               lam_ref, out_ref):
    lam = lam_ref[0, 0]
    wu = wu_ref[...]
    wd = wd_ref[...]
    wl = wl_ref[...]
    wr = wr_ref[...]
    deg = deg_ref[...]
    BH = B * H

    E = (jax.lax.broadcasted_iota(jnp.int32, (H, HC), 0) // S
         == jax.lax.broadcasted_iota(jnp.int32, (H, HC), 1)).astype(jnp.float32)
    Et = (jax.lax.broadcasted_iota(jnp.int32, (HC, H), 1) // S
          == jax.lax.broadcasted_iota(jnp.int32, (HC, H), 0)).astype(jnp.float32)
    E4 = (jax.lax.broadcasted_iota(jnp.int32, (BH, B * HC), 0) // S
          == jax.lax.broadcasted_iota(jnp.int32, (BH, B * HC), 1)).astype(jnp.float32)
    E4t = (jax.lax.broadcasted_iota(jnp.int32, (B * HC, BH), 1) // S
           == jax.lax.broadcasted_iota(jnp.int32, (B * HC, BH), 0)).astype(jnp.float32)
    inv = 1.0 / float(S * S)
    msk = mask_ref[...]

    zrow = jnp.zeros((1, W), jnp.float32)
    zcol = jnp.zeros((BH, 1), jnp.float32)

    def A_op(y):
        nu = jnp.concatenate([zrow, y[: BH - 1, :]], axis=0)
        nd = jnp.concatenate([y[1:, :], zrow], axis=0)
        nl = jnp.concatenate([zcol, y[:, : W - 1]], axis=1)
        nr = jnp.concatenate([y[:, 1:], zcol], axis=1)
        Ly = deg * y - (wu * nu + wd * nd + wl * nl + wr * nr)
        d = jax.lax.dot(E4t, jax.lax.dot(y, E)) * inv
        u = jax.lax.dot(jax.lax.dot(E4, msk * d), Et) * inv
        return Ly + lam * u

    b = lam * (jax.lax.dot(jax.lax.dot(E4, msk * src_ref[...]), Et) * inv)
    x = jax.lax.dot(jax.lax.dot(E4, src_ref[...]), Et)

    r = b - A_op(x)
    p = r
    rs = jnp.sum(r * r)

    def body(_, carry):
        x, r, p, rs = carry
        Ap = A_op(p)
        alpha = rs / (jnp.sum(p * Ap) + 1e-12)
        x = x + alpha * p
        r = r - alpha * Ap
        rs_new = jnp.sum(r * r)
        p = r + (rs_new / (rs + 1e-12)) * p
        return x, r, p, rs_new

    x, r, p, rs = jax.lax.fori_loop(0, NIT, body, (x, r, p, rs))
    out_ref[...] = x


def kernel(guide, source, mask_lr, y_bicubic, var_w, var_b, fe_w1, fe_b1,
           fe_w2, fe_b2, fe_w3, fe_b3, log_lambda, log_mu):
    mu = jnp.exp(log_mu).reshape(1, 1)
    lam = jnp.exp(log_lambda).reshape(1, 1)

    g_f = guide.reshape(B, 3, N)
    yb_f = y_bicubic.reshape(B, 1, N)

    w1c = jnp.concatenate([fe_w1, var_w], axis=0)
    w1_flat = w1c.transpose(0, 2, 3, 1).reshape(33, 36)
    b1c = jnp.concatenate([fe_b1, var_b], axis=0).reshape(33, 1)
    w2r = fe_w2.transpose(0, 2, 3, 1).reshape(32, 288)
    w3r = fe_w3.transpose(0, 2, 3, 1).reshape(32, 288)
    b2 = fe_b2.reshape(32, 1)
    b3 = fe_b3.reshape(32, 1)

    var_f, aff_f = pl.pallas_call(
        _conv_kernel,
        grid=(B,),
        in_specs=[
            pl.BlockSpec((1, 3, N), lambda b: (b, 0, 0)),
            pl.BlockSpec((1, 1, N), lambda b: (b, 0, 0)),
            pl.BlockSpec((33, 36), lambda b: (0, 0)),
            pl.BlockSpec((33, 1), lambda b: (0, 0)),
            pl.BlockSpec((32, 288), lambda b: (0, 0)),
            pl.BlockSpec((32, 1), lambda b: (0, 0)),
            pl.BlockSpec((32, 288), lambda b: (0, 0)),
            pl.BlockSpec((32, 1), lambda b: (0, 0)),
            pl.BlockSpec((1, 1), lambda b: (0, 0)),
        ],
        out_specs=[
            pl.BlockSpec((1, 1, N), lambda b: (b, 0, 0)),
            pl.BlockSpec((1, 5, N), lambda b: (b, 0, 0)),
        ],
        out_shape=[
            jax.ShapeDtypeStruct((B, 1, N), jnp.float32),
            jax.ShapeDtypeStruct((B, 5, N), jnp.float32),
        ],
    )(g_f, yb_f, w1_flat, b1c, w2r, b2, w3r, b3, mu)

    var = var_f.reshape(B, 1, H, W)
    aff = aff_f.reshape(B, 5, H, W)

    src = source.reshape(B * HC, HC)
    msk = mask_lr.reshape(B * HC, HC)
    BH = B * H

    y = pl.pallas_call(
        _cg_kernel,
        grid=(1,),
        in_specs=[
            pl.BlockSpec((BH, W), lambda i: (0, 0)),
            pl.BlockSpec((BH, W), lambda i: (0, 0)),
            pl.BlockSpec((BH, W), lambda i: (0, 0)),
            pl.BlockSpec((BH, W), lambda i: (0, 0)),
            pl.BlockSpec((BH, W), lambda i: (0, 0)),
            pl.BlockSpec((B * HC, HC), lambda i: (0, 0)),
            pl.BlockSpec((B * HC, HC), lambda i: (0, 0)),
            pl.BlockSpec((1, 1), lambda i: (0, 0)),
        ],
        out_specs=pl.BlockSpec((BH, W), lambda i: (0, 0)),
        out_shape=jax.ShapeDtypeStruct((BH, W), jnp.float32),
    )(aff_f[:, 0].reshape(BH, W), aff_f[:, 1].reshape(BH, W),
      aff_f[:, 2].reshape(BH, W), aff_f[:, 3].reshape(BH, W),
      aff_f[:, 4].reshape(BH, W), src, msk, lam)

    return (y.reshape(B, 1, H, W), var, aff)


# speedup vs baseline: 1.1748x; 1.1748x over previous
import jax
import jax.numpy as jnp
from jax.experimental import pallas as pl
from jax.experimental.pallas import tpu as pltpu

H = 256
W = 256
N = H * W
S = 8
HC = H // S
B = 4
NIT = 30


def _shift_flat(x, dh, dw, col_ids):
    s = W * dh + dw
    C = x.shape[0]
    if s > 0:
        x = jnp.concatenate([jnp.zeros((C, s), jnp.float32), x[:, : N - s]], axis=1)
    elif s < 0:
        x = jnp.concatenate([x[:, -s:], jnp.zeros((C, -s), jnp.float32)], axis=1)
    if dw == 1:
        x = jnp.where(col_ids == 0, 0.0, x)
    elif dw == -1:
        x = jnp.where(col_ids == W - 1, 0.0, x)
    return x


def _conv_kernel(g_ref, yb_ref, w1_ref, b1_ref, w2_ref, b2_ref, w3_ref, b3_ref,
                 mu_ref, var_ref, aff_ref):
    col_ids = jax.lax.broadcasted_iota(jnp.int32, (1, N), 1) % W
    x0 = jnp.concatenate([g_ref[0], yb_ref[0]], axis=0)

    cols = [_shift_flat(x0, 1 - i, 1 - j, col_ids) for i in range(3) for j in range(3)]
    im2col = jnp.concatenate(cols, axis=0)
    l1 = jax.lax.dot(w1_ref[...], im2col, preferred_element_type=jnp.float32)
    l1 = l1 + b1_ref[...]
    var_ref[0] = l1[32:33]
    f = jnp.maximum(l1[:32], 0.0)

    CH = 16384
    PAD = 512
    for w_ref, b_ref, relu in ((w2_ref, b2_ref, True), (w3_ref, b3_ref, False)):
        fpad = jnp.concatenate(
            [jnp.zeros((32, PAD), jnp.float32), f,
             jnp.zeros((32, PAD), jnp.float32)], axis=1)
        outs = []
        for c in range(N // CH):
            base = c * CH
            cols = []
            for i in range(3):
                for j in range(3):
                    s = W * (1 - i) + (1 - j)
                    st = base + PAD - s
                    sl = fpad[:, st:st + CH]
                    cc = col_ids[:, :CH]
                    if j == 0:
                        sl = jnp.where(cc == 0, 0.0, sl)
                    elif j == 2:
                        sl = jnp.where(cc == W - 1, 0.0, sl)
                    cols.append(sl)
            im2col_c = jnp.concatenate(cols, axis=0)
            outs.append(jax.lax.dot(w_ref[...], im2col_c,
                                    preferred_element_type=jnp.float32))
        f = jnp.concatenate(outs, axis=1) + b_ref[...]
        if relu:
            f = jnp.maximum(f, 0.0)

    mu = mu_ref[0, 0]
    row_ids = jax.lax.broadcasted_iota(jnp.int32, (1, N), 1) // W

    def aff(dh, dw, border_ids, border_val):
        fn = _shift_flat(f, dh, dw, col_ids)
        d2 = jnp.sum((f - fn) ** 2, axis=0, keepdims=True)
        wdir = jnp.exp(-d2 / mu)
        return jnp.where(border_ids == border_val, 0.0, wdir)

    wu = aff(1, 0, row_ids, 0)
    wd = aff(-1, 0, row_ids, H - 1)
    wl = aff(0, 1, col_ids, 0)
    wr = aff(0, -1, col_ids, W - 1)
    deg = wu + wd + wl + wr
    aff_ref[0] = jnp.concatenate([wu, wd, wl, wr, deg], axis=0)


def _cg_kernel(wu_ref, wd_ref, wl_ref, wr_ref, deg_ref, src_ref, mask_ref,
               lam_ref, out_ref):
    lam = lam_ref[0, 0]
    wu = wu_ref[...]
    wd = wd_ref[...]
    wl = wl_ref[...]
    wr = wr_ref[...]
    deg = deg_ref[...]
    E = (jax.lax.broadcasted_iota(jnp.int32, (H, HC), 0) // S
         == jax.lax.broadcasted_iota(jnp.int32, (H, HC), 1)).astype(jnp.float32)
    Et = (jax.lax.broadcasted_iota(jnp.int32, (HC, H), 1) // S
          == jax.lax.broadcasted_iota(jnp.int32, (HC, H), 0)).astype(jnp.float32)
    inv = 1.0 / float(S * S)

    zrow = jnp.zeros((B, 1, W), jnp.float32)
    zcol = jnp.zeros((B, H, 1), jnp.float32)

    def A_op(y):
        nu = jnp.concatenate([zrow, y[:, : H - 1, :]], axis=1)
        nd = jnp.concatenate([y[:, 1:, :], zrow], axis=1)
        nl = jnp.concatenate([zcol, y[:, :, : W - 1]], axis=2)
        nr = jnp.concatenate([y[:, :, 1:], zcol], axis=2)
        Ly = deg * y - (wu * nu + wd * nd + wl * nl + wr * nr)
        ups = []
        for k in range(B):
            dk = jax.lax.dot(jax.lax.dot(Et, y[k]), E) * inv
            zk = mask_ref[k] * dk
            ups.append((jax.lax.dot(jax.lax.dot(E, zk), Et) * inv).reshape(1, H, W))
        return Ly + lam * jnp.concatenate(ups, axis=0)

    bs = []
    x0s = []
    for k in range(B):
        ms = mask_ref[k] * src_ref[k]
        bs.append((jax.lax.dot(jax.lax.dot(E, ms), Et) * inv).reshape(1, H, W))
        x0s.append(jax.lax.dot(jax.lax.dot(E, src_ref[k]), Et).reshape(1, H, W))
    b = lam * jnp.concatenate(bs, axis=0)
    x = jnp.concatenate(x0s, axis=0)

    r = b - A_op(x)
    p = r
    rs = jnp.sum(r * r)

    def body(_, carry):
        x, r, p, rs = carry
        Ap = A_op(p)
        alpha = rs / (jnp.sum(p * Ap) + 1e-12)
        x = x + alpha * p
        r = r - alpha * Ap
        rs_new = jnp.sum(r * r)
        p = r + (rs_new / (rs + 1e-12)) * p
        return x, r, p, rs_new

    x, r, p, rs = jax.lax.fori_loop(0, NIT, body, (x, r, p, rs))
    out_ref[...] = x


def kernel(guide, source, mask_lr, y_bicubic, var_w, var_b, fe_w1, fe_b1,
           fe_w2, fe_b2, fe_w3, fe_b3, log_lambda, log_mu):
    mu = jnp.exp(log_mu).reshape(1, 1)
    lam = jnp.exp(log_lambda).reshape(1, 1)

    g_f = guide.reshape(B, 3, N)
    yb_f = y_bicubic.reshape(B, 1, N)

    w1c = jnp.concatenate([fe_w1, var_w], axis=0)
    w1_flat = w1c.transpose(0, 2, 3, 1).reshape(33, 36)
    b1c = jnp.concatenate([fe_b1, var_b], axis=0).reshape(33, 1)
    w2r = fe_w2.transpose(0, 2, 3, 1).reshape(32, 288)
    w3r = fe_w3.transpose(0, 2, 3, 1).reshape(32, 288)
    b2 = fe_b2.reshape(32, 1)
    b3 = fe_b3.reshape(32, 1)

    var_f, aff_f = pl.pallas_call(
        _conv_kernel,
        grid=(B,),
        in_specs=[
            pl.BlockSpec((1, 3, N), lambda b: (b, 0, 0)),
            pl.BlockSpec((1, 1, N), lambda b: (b, 0, 0)),
            pl.BlockSpec((33, 36), lambda b: (0, 0)),
            pl.BlockSpec((33, 1), lambda b: (0, 0)),
            pl.BlockSpec((32, 288), lambda b: (0, 0)),
            pl.BlockSpec((32, 1), lambda b: (0, 0)),
            pl.BlockSpec((32, 288), lambda b: (0, 0)),
            pl.BlockSpec((32, 1), lambda b: (0, 0)),
            pl.BlockSpec((1, 1), lambda b: (0, 0)),
        ],
        out_specs=[
            pl.BlockSpec((1, 1, N), lambda b: (b, 0, 0)),
            pl.BlockSpec((1, 5, N), lambda b: (b, 0, 0)),
        ],
        out_shape=[
            jax.ShapeDtypeStruct((B, 1, N), jnp.float32),
            jax.ShapeDtypeStruct((B, 5, N), jnp.float32),
        ],
        compiler_params=pltpu.CompilerParams(
            dimension_semantics=("parallel",)),
    )(g_f, yb_f, w1_flat, b1c, w2r, b2, w3r, b3, mu)

    var = var_f.reshape(B, 1, H, W)
    aff = aff_f.reshape(B, 5, H, W)

    src = source.reshape(B, HC, HC)
    msk = mask_lr.reshape(B, HC, HC)
    aff3 = aff_f.reshape(B, 5, H, W)

    y = pl.pallas_call(
        _cg_kernel,
        grid=(1,),
        in_specs=[
            pl.BlockSpec((B, H, W), lambda i: (0, 0, 0)),
            pl.BlockSpec((B, H, W), lambda i: (0, 0, 0)),
            pl.BlockSpec((B, H, W), lambda i: (0, 0, 0)),
            pl.BlockSpec((B, H, W), lambda i: (0, 0, 0)),
            pl.BlockSpec((B, H, W), lambda i: (0, 0, 0)),
            pl.BlockSpec((B, HC, HC), lambda i: (0, 0, 0)),
            pl.BlockSpec((B, HC, HC), lambda i: (0, 0, 0)),
            pl.BlockSpec((1, 1), lambda i: (0, 0)),
        ],
        out_specs=pl.BlockSpec((B, H, W), lambda i: (0, 0, 0)),
        out_shape=jax.ShapeDtypeStruct((B, H, W), jnp.float32),
    )(aff3[:, 0], aff3[:, 1], aff3[:, 2], aff3[:, 3], aff3[:, 4],
      src, msk, lam)

    return (y.reshape(B, 1, H, W), var, aff)


# speedup vs baseline: 1.2105x; 1.0304x over previous
import jax
import jax.numpy as jnp
from jax.experimental import pallas as pl
from jax.experimental.pallas import tpu as pltpu

H = 256
W = 256
N = H * W
S = 8
HC = H // S
B = 4
NIT = 30


def _shift_flat(x, dh, dw, col_ids):
    s = W * dh + dw
    C = x.shape[0]
    if s > 0:
        x = jnp.concatenate([jnp.zeros((C, s), jnp.float32), x[:, : N - s]], axis=1)
    elif s < 0:
        x = jnp.concatenate([x[:, -s:], jnp.zeros((C, -s), jnp.float32)], axis=1)
    if dw == 1:
        x = jnp.where(col_ids == 0, 0.0, x)
    elif dw == -1:
        x = jnp.where(col_ids == W - 1, 0.0, x)
    return x


def _conv_kernel(g_ref, yb_ref, w1_ref, b1_ref, w2_ref, b2_ref, w3_ref, b3_ref,
                 mu_ref, var_ref, aff_ref):
    col_ids = jax.lax.broadcasted_iota(jnp.int32, (1, N), 1) % W
    x0 = jnp.concatenate([g_ref[0], yb_ref[0]], axis=0)

    cols = [_shift_flat(x0, 1 - i, 1 - j, col_ids) for i in range(3) for j in range(3)]
    im2col = jnp.concatenate(cols, axis=0)
    l1 = jax.lax.dot(w1_ref[...], im2col, preferred_element_type=jnp.float32)
    l1 = l1 + b1_ref[...]
    var_ref[0] = l1[32:33]
    f = jnp.maximum(l1[:32], 0.0)

    CH = 16384
    PAD = 512
    for w_ref, b_ref, relu in ((w2_ref, b2_ref, True), (w3_ref, b3_ref, False)):
        fpad = jnp.concatenate(
            [jnp.zeros((32, PAD), jnp.float32), f,
             jnp.zeros((32, PAD), jnp.float32)], axis=1)
        outs = []
        for c in range(N // CH):
            base = c * CH
            cols = []
            for i in range(3):
                for j in range(3):
                    s = W * (1 - i) + (1 - j)
                    st = base + PAD - s
                    sl = fpad[:, st:st + CH]
                    cc = col_ids[:, :CH]
                    if j == 0:
                        sl = jnp.where(cc == 0, 0.0, sl)
                    elif j == 2:
                        sl = jnp.where(cc == W - 1, 0.0, sl)
                    cols.append(sl)
            im2col_c = jnp.concatenate(cols, axis=0)
            outs.append(jax.lax.dot(w_ref[...], im2col_c,
                                    preferred_element_type=jnp.float32))
        f = jnp.concatenate(outs, axis=1) + b_ref[...]
        if relu:
            f = jnp.maximum(f, 0.0)

    mu = mu_ref[0, 0]
    row_ids = jax.lax.broadcasted_iota(jnp.int32, (1, N), 1) // W
    s = jnp.sum(f * f, axis=0, keepdims=True)

    def aff(dh, dw, border_ids, border_val):
        fn = _shift_flat(f, dh, dw, col_ids)
        cross = jnp.sum(f * fn, axis=0, keepdims=True)
        d2 = s + _shift_flat(s, dh, dw, col_ids) - 2.0 * cross
        wdir = jnp.exp(-d2 / mu)
        return jnp.where(border_ids == border_val, 0.0, wdir)

    wu = aff(1, 0, row_ids, 0)
    wl = aff(0, 1, col_ids, 0)
    wd = jnp.concatenate([wu[:, W:], jnp.zeros((1, W), jnp.float32)], axis=1)
    wr = jnp.concatenate([wl[:, 1:], jnp.zeros((1, 1), jnp.float32)], axis=1)
    deg = wu + wd + wl + wr
    aff_ref[0] = jnp.concatenate([wu, wd, wl, wr, deg], axis=0)


def _cg_kernel(wu_ref, wd_ref, wl_ref, wr_ref, deg_ref, src_ref, mask_ref,
               lam_ref, out_ref):
    lam = lam_ref[0, 0]
    wu = wu_ref[...]
    wd = wd_ref[...]
    wl = wl_ref[...]
    wr = wr_ref[...]
    deg = deg_ref[...]
    E = (jax.lax.broadcasted_iota(jnp.int32, (H, HC), 0) // S
         == jax.lax.broadcasted_iota(jnp.int32, (H, HC), 1)).astype(jnp.float32)
    Et = (jax.lax.broadcasted_iota(jnp.int32, (HC, H), 1) // S
          == jax.lax.broadcasted_iota(jnp.int32, (HC, H), 0)).astype(jnp.float32)
    inv = 1.0 / float(S * S)

    zrow = jnp.zeros((B, 1, W), jnp.float32)
    zcol = jnp.zeros((B, H, 1), jnp.float32)

    def A_op(y):
        nu = jnp.concatenate([zrow, y[:, : H - 1, :]], axis=1)
        nd = jnp.concatenate([y[:, 1:, :], zrow], axis=1)
        nl = jnp.concatenate([zcol, y[:, :, : W - 1]], axis=2)
        nr = jnp.concatenate([y[:, :, 1:], zcol], axis=2)
        Ly = deg * y - (wu * nu + wd * nd + wl * nl + wr * nr)
        ups = []
        for k in range(B):
            dk = jax.lax.dot(jax.lax.dot(Et, y[k]), E) * inv
            zk = mask_ref[k] * dk
            ups.append((jax.lax.dot(jax.lax.dot(E, zk), Et) * inv).reshape(1, H, W))
        return Ly + lam * jnp.concatenate(ups, axis=0)

    bs = []
    x0s = []
    for k in range(B):
        ms = mask_ref[k] * src_ref[k]
        bs.append((jax.lax.dot(jax.lax.dot(E, ms), Et) * inv).reshape(1, H, W))
        x0s.append(jax.lax.dot(jax.lax.dot(E, src_ref[k]), Et).reshape(1, H, W))
    b = lam * jnp.concatenate(bs, axis=0)
    x = jnp.concatenate(x0s, axis=0)

    r = b - A_op(x)
    p = r
    rs = jnp.sum(r * r)

    def body(_, carry):
        x, r, p, rs = carry
        Ap = A_op(p)
        alpha = rs / (jnp.sum(p * Ap) + 1e-12)
        x = x + alpha * p
        r = r - alpha * Ap
        rs_new = jnp.sum(r * r)
        p = r + (rs_new / (rs + 1e-12)) * p
        return x, r, p, rs_new

    x, r, p, rs = jax.lax.fori_loop(0, NIT, body, (x, r, p, rs))
    out_ref[...] = x


def kernel(guide, source, mask_lr, y_bicubic, var_w, var_b, fe_w1, fe_b1,
           fe_w2, fe_b2, fe_w3, fe_b3, log_lambda, log_mu):
    mu = jnp.exp(log_mu).reshape(1, 1)
    lam = jnp.exp(log_lambda).reshape(1, 1)

    g_f = guide.reshape(B, 3, N)
    yb_f = y_bicubic.reshape(B, 1, N)

    w1c = jnp.concatenate([fe_w1, var_w], axis=0)
    w1_flat = w1c.transpose(0, 2, 3, 1).reshape(33, 36)
    b1c = jnp.concatenate([fe_b1, var_b], axis=0).reshape(33, 1)
    w2r = fe_w2.transpose(0, 2, 3, 1).reshape(32, 288)
    w3r = fe_w3.transpose(0, 2, 3, 1).reshape(32, 288)
    b2 = fe_b2.reshape(32, 1)
    b3 = fe_b3.reshape(32, 1)

    var_f, aff_f = pl.pallas_call(
        _conv_kernel,
        grid=(B,),
        in_specs=[
            pl.BlockSpec((1, 3, N), lambda b: (b, 0, 0)),
            pl.BlockSpec((1, 1, N), lambda b: (b, 0, 0)),
            pl.BlockSpec((33, 36), lambda b: (0, 0)),
            pl.BlockSpec((33, 1), lambda b: (0, 0)),
            pl.BlockSpec((32, 288), lambda b: (0, 0)),
            pl.BlockSpec((32, 1), lambda b: (0, 0)),
            pl.BlockSpec((32, 288), lambda b: (0, 0)),
            pl.BlockSpec((32, 1), lambda b: (0, 0)),
            pl.BlockSpec((1, 1), lambda b: (0, 0)),
        ],
        out_specs=[
            pl.BlockSpec((1, 1, N), lambda b: (b, 0, 0)),
            pl.BlockSpec((1, 5, N), lambda b: (b, 0, 0)),
        ],
        out_shape=[
            jax.ShapeDtypeStruct((B, 1, N), jnp.float32),
            jax.ShapeDtypeStruct((B, 5, N), jnp.float32),
        ],
        compiler_params=pltpu.CompilerParams(
            dimension_semantics=("parallel",)),
    )(g_f, yb_f, w1_flat, b1c, w2r, b2, w3r, b3, mu)

    var = var_f.reshape(B, 1, H, W)
    aff = aff_f.reshape(B, 5, H, W)

    src = source.reshape(B, HC, HC)
    msk = mask_lr.reshape(B, HC, HC)
    aff3 = aff_f.reshape(B, 5, H, W)

    y = pl.pallas_call(
        _cg_kernel,
        grid=(1,),
        in_specs=[
            pl.BlockSpec((B, H, W), lambda i: (0, 0, 0)),
            pl.BlockSpec((B, H, W), lambda i: (0, 0, 0)),
            pl.BlockSpec((B, H, W), lambda i: (0, 0, 0)),
            pl.BlockSpec((B, H, W), lambda i: (0, 0, 0)),
            pl.BlockSpec((B, H, W), lambda i: (0, 0, 0)),
            pl.BlockSpec((B, HC, HC), lambda i: (0, 0, 0)),
            pl.BlockSpec((B, HC, HC), lambda i: (0, 0, 0)),
            pl.BlockSpec((1, 1), lambda i: (0, 0)),
        ],
        out_specs=pl.BlockSpec((B, H, W), lambda i: (0, 0, 0)),
        out_shape=jax.ShapeDtypeStruct((B, H, W), jnp.float32),
    )(aff3[:, 0], aff3[:, 1], aff3[:, 2], aff3[:, 3], aff3[:, 4],
      src, msk, lam)

    return (y.reshape(B, 1, H, W), var, aff)


# speedup vs baseline: 1.2321x; 1.0178x over previous
import jax
import jax.numpy as jnp
from jax.experimental import pallas as pl
from jax.experimental.pallas import tpu as pltpu

H = 256
W = 256
N = H * W
S = 8
HC = H // S
B = 4
NIT = 30


def _shift_flat(x, dh, dw, col_ids):
    s = W * dh + dw
    C = x.shape[0]
    if s > 0:
        x = jnp.concatenate([jnp.zeros((C, s), jnp.float32), x[:, : N - s]], axis=1)
    elif s < 0:
        x = jnp.concatenate([x[:, -s:], jnp.zeros((C, -s), jnp.float32)], axis=1)
    if dw == 1:
        x = jnp.where(col_ids == 0, 0.0, x)
    elif dw == -1:
        x = jnp.where(col_ids == W - 1, 0.0, x)
    return x


def _conv_kernel(g_ref, yb_ref, w1_ref, b1_ref, w2_ref, b2_ref, w3_ref, b3_ref,
                 mu_ref, var_ref, aff_ref):
    col_ids = jax.lax.broadcasted_iota(jnp.int32, (1, N), 1) % W
    x0 = jnp.concatenate([g_ref[0], yb_ref[0]], axis=0)

    cols = [_shift_flat(x0, 1 - i, 1 - j, col_ids) for i in range(3) for j in range(3)]
    im2col = jnp.concatenate(cols, axis=0)
    l1 = jax.lax.dot(w1_ref[...], im2col, preferred_element_type=jnp.float32)
    l1 = l1 + b1_ref[...]
    var_ref[0] = l1[32:33]
    f = jnp.maximum(l1[:32], 0.0)

    CH = 16384
    PAD = 512
    for w_ref, b_ref, relu in ((w2_ref, b2_ref, True), (w3_ref, b3_ref, False)):
        fpad = jnp.concatenate(
            [jnp.zeros((32, PAD), jnp.float32), f,
             jnp.zeros((32, PAD), jnp.float32)], axis=1)
        outs = []
        for c in range(N // CH):
            base = c * CH
            cols = []
            for i in range(3):
                for j in range(3):
                    s = W * (1 - i) + (1 - j)
                    st = base + PAD - s
                    sl = fpad[:, st:st + CH]
                    cc = col_ids[:, :CH]
                    if j == 0:
                        sl = jnp.where(cc == 0, 0.0, sl)
                    elif j == 2:
                        sl = jnp.where(cc == W - 1, 0.0, sl)
                    cols.append(sl)
            im2col_c = jnp.concatenate(cols, axis=0)
            outs.append(jax.lax.dot(w_ref[...], im2col_c,
                                    preferred_element_type=jnp.float32))
        f = jnp.concatenate(outs, axis=1) + b_ref[...]
        if relu:
            f = jnp.maximum(f, 0.0)

    mu = mu_ref[0, 0]
    row_ids = jax.lax.broadcasted_iota(jnp.int32, (1, N), 1) // W
    s = jnp.sum(f * f, axis=0, keepdims=True)

    def aff(dh, dw, border_ids, border_val):
        fn = _shift_flat(f, dh, dw, col_ids)
        cross = jnp.sum(f * fn, axis=0, keepdims=True)
        d2 = s + _shift_flat(s, dh, dw, col_ids) - 2.0 * cross
        wdir = jnp.exp(-d2 / mu)
        return jnp.where(border_ids == border_val, 0.0, wdir)

    wu = aff(1, 0, row_ids, 0)
    wl = aff(0, 1, col_ids, 0)
    wd = jnp.concatenate([wu[:, W:], jnp.zeros((1, W), jnp.float32)], axis=1)
    wr = jnp.concatenate([wl[:, 1:], jnp.zeros((1, 1), jnp.float32)], axis=1)
    deg = wu + wd + wl + wr
    aff_ref[0] = jnp.concatenate([wu, wd, wl, wr, deg], axis=0)


def _cg_kernel(aff_ref, src_ref, mask_ref, lam_ref, out_ref):
    lam = lam_ref[0, 0]
    wu = aff_ref[:, 0]
    wd = aff_ref[:, 1]
    wl = aff_ref[:, 2]
    wr = aff_ref[:, 3]
    deg = aff_ref[:, 4]
    E = (jax.lax.broadcasted_iota(jnp.int32, (H, HC), 0) // S
         == jax.lax.broadcasted_iota(jnp.int32, (H, HC), 1)).astype(jnp.float32)
    Et = (jax.lax.broadcasted_iota(jnp.int32, (HC, H), 1) // S
          == jax.lax.broadcasted_iota(jnp.int32, (HC, H), 0)).astype(jnp.float32)
    inv = 1.0 / float(S * S)

    zrow = jnp.zeros((B, 1, W), jnp.float32)
    zcol = jnp.zeros((B, H, 1), jnp.float32)

    def A_op(y):
        nu = jnp.concatenate([zrow, y[:, : H - 1, :]], axis=1)
        nd = jnp.concatenate([y[:, 1:, :], zrow], axis=1)
        nl = jnp.concatenate([zcol, y[:, :, : W - 1]], axis=2)
        nr = jnp.concatenate([y[:, :, 1:], zcol], axis=2)
        Ly = deg * y - (wu * nu + wd * nd + wl * nl + wr * nr)
        ups = []
        for k in range(B):
            dk = jax.lax.dot(jax.lax.dot(Et, y[k]), E) * inv
            zk = mask_ref[k] * dk
            ups.append((jax.lax.dot(jax.lax.dot(E, zk), Et) * inv).reshape(1, H, W))
        return Ly + lam * jnp.concatenate(ups, axis=0)

    bs = []
    x0s = []
    for k in range(B):
        ms = mask_ref[k] * src_ref[k]
        bs.append((jax.lax.dot(jax.lax.dot(E, ms), Et) * inv).reshape(1, H, W))
        x0s.append(jax.lax.dot(jax.lax.dot(E, src_ref[k]), Et).reshape(1, H, W))
    b = lam * jnp.concatenate(bs, axis=0)
    x = jnp.concatenate(x0s, axis=0)

    r = b - A_op(x)
    p = r
    rs = jnp.sum(r * r)

    def body(_, carry):
        x, r, p, rs = carry
        Ap = A_op(p)
        alpha = rs / (jnp.sum(p * Ap) + 1e-12)
        x = x + alpha * p
        r = r - alpha * Ap
        rs_new = jnp.sum(r * r)
        p = r + (rs_new / (rs + 1e-12)) * p
        return x, r, p, rs_new

    x, r, p, rs = jax.lax.fori_loop(0, NIT, body, (x, r, p, rs))
    out_ref[...] = x


def kernel(guide, source, mask_lr, y_bicubic, var_w, var_b, fe_w1, fe_b1,
           fe_w2, fe_b2, fe_w3, fe_b3, log_lambda, log_mu):
    mu = jnp.exp(log_mu).reshape(1, 1)
    lam = jnp.exp(log_lambda).reshape(1, 1)

    g_f = guide.reshape(B, 3, N)
    yb_f = y_bicubic.reshape(B, 1, N)

    w1c = jnp.concatenate([fe_w1, var_w], axis=0)
    w1_flat = w1c.transpose(0, 2, 3, 1).reshape(33, 36)
    b1c = jnp.concatenate([fe_b1, var_b], axis=0).reshape(33, 1)
    w2r = fe_w2.transpose(0, 2, 3, 1).reshape(32, 288)
    w3r = fe_w3.transpose(0, 2, 3, 1).reshape(32, 288)
    b2 = fe_b2.reshape(32, 1)
    b3 = fe_b3.reshape(32, 1)

    var_f, aff_f = pl.pallas_call(
        _conv_kernel,
        grid=(B,),
        in_specs=[
            pl.BlockSpec((1, 3, N), lambda b: (b, 0, 0)),
            pl.BlockSpec((1, 1, N), lambda b: (b, 0, 0)),
            pl.BlockSpec((33, 36), lambda b: (0, 0)),
            pl.BlockSpec((33, 1), lambda b: (0, 0)),
            pl.BlockSpec((32, 288), lambda b: (0, 0)),
            pl.BlockSpec((32, 1), lambda b: (0, 0)),
            pl.BlockSpec((32, 288), lambda b: (0, 0)),
            pl.BlockSpec((32, 1), lambda b: (0, 0)),
            pl.BlockSpec((1, 1), lambda b: (0, 0)),
        ],
        out_specs=[
            pl.BlockSpec((1, 1, N), lambda b: (b, 0, 0)),
            pl.BlockSpec((1, 5, N), lambda b: (b, 0, 0)),
        ],
        out_shape=[
            jax.ShapeDtypeStruct((B, 1, N), jnp.float32),
            jax.ShapeDtypeStruct((B, 5, N), jnp.float32),
        ],
        compiler_params=pltpu.CompilerParams(
            dimension_semantics=("parallel",)),
    )(g_f, yb_f, w1_flat, b1c, w2r, b2, w3r, b3, mu)

    var = var_f.reshape(B, 1, H, W)
    aff = aff_f.reshape(B, 5, H, W)

    src = source.reshape(B, HC, HC)
    msk = mask_lr.reshape(B, HC, HC)
    aff3 = aff_f.reshape(B, 5, H, W)

    y = pl.pallas_call(
        _cg_kernel,
        grid=(1,),
        in_specs=[
            pl.BlockSpec((B, 5, H, W), lambda i: (0, 0, 0, 0)),
            pl.BlockSpec((B, HC, HC), lambda i: (0, 0, 0)),
            pl.BlockSpec((B, HC, HC), lambda i: (0, 0, 0)),
            pl.BlockSpec((1, 1), lambda i: (0, 0)),
        ],
        out_specs=pl.BlockSpec((B, H, W), lambda i: (0, 0, 0)),
        out_shape=jax.ShapeDtypeStruct((B, H, W), jnp.float32),
    )(aff3, src, msk, lam)

    return (y.reshape(B, 1, H, W), var, aff)


# speedup vs baseline: 1.2382x; 1.0050x over previous
import jax
import jax.numpy as jnp
from jax.experimental import pallas as pl
from jax.experimental.pallas import tpu as pltpu

H = 256
W = 256
N = H * W
S = 8
HC = H // S
B = 4
NIT = 30


def _shift_flat(x, dh, dw, col_ids):
    s = W * dh + dw
    C = x.shape[0]
    if s > 0:
        x = jnp.concatenate([jnp.zeros((C, s), jnp.float32), x[:, : N - s]], axis=1)
    elif s < 0:
        x = jnp.concatenate([x[:, -s:], jnp.zeros((C, -s), jnp.float32)], axis=1)
    if dw == 1:
        x = jnp.where(col_ids == 0, 0.0, x)
    elif dw == -1:
        x = jnp.where(col_ids == W - 1, 0.0, x)
    return x


def _conv_kernel(g_ref, yb_ref, w1_ref, b1_ref, w2_ref, b2_ref, w3_ref, b3_ref,
                 mu_ref, var_ref, aff_ref):
    col_ids = jax.lax.broadcasted_iota(jnp.int32, (1, N), 1) % W
    x0 = jnp.concatenate([g_ref[0], yb_ref[0]], axis=0)

    cols = [_shift_flat(x0, 1 - i, 1 - j, col_ids) for i in range(3) for j in range(3)]
    im2col = jnp.concatenate(cols, axis=0)
    l1 = jax.lax.dot(w1_ref[...], im2col, preferred_element_type=jnp.float32)
    l1 = l1 + b1_ref[...]
    var_ref[0] = l1[32:33]
    f = jnp.maximum(l1[:32], 0.0)

    CH = 8192
    PAD = 512
    for w_ref, b_ref, relu in ((w2_ref, b2_ref, True), (w3_ref, b3_ref, False)):
        fpad = jnp.concatenate(
            [jnp.zeros((32, PAD), jnp.float32), f,
             jnp.zeros((32, PAD), jnp.float32)], axis=1)
        outs = []
        for c in range(N // CH):
            base = c * CH
            cols = []
            for i in range(3):
                for j in range(3):
                    s = W * (1 - i) + (1 - j)
                    st = base + PAD - s
                    sl = fpad[:, st:st + CH]
                    cc = col_ids[:, :CH]
                    if j == 0:
                        sl = jnp.where(cc == 0, 0.0, sl)
                    elif j == 2:
                        sl = jnp.where(cc == W - 1, 0.0, sl)
                    cols.append(sl)
            im2col_c = jnp.concatenate(cols, axis=0)
            outs.append(jax.lax.dot(w_ref[...], im2col_c,
                                    preferred_element_type=jnp.float32))
        f = jnp.concatenate(outs, axis=1) + b_ref[...]
        if relu:
            f = jnp.maximum(f, 0.0)

    mu = mu_ref[0, 0]
    row_ids = jax.lax.broadcasted_iota(jnp.int32, (1, N), 1) // W
    s = jnp.sum(f * f, axis=0, keepdims=True)

    def aff(dh, dw, border_ids, border_val):
        fn = _shift_flat(f, dh, dw, col_ids)
        cross = jnp.sum(f * fn, axis=0, keepdims=True)
        d2 = s + _shift_flat(s, dh, dw, col_ids) - 2.0 * cross
        wdir = jnp.exp(-d2 / mu)
        return jnp.where(border_ids == border_val, 0.0, wdir)

    wu = aff(1, 0, row_ids, 0)
    wl = aff(0, 1, col_ids, 0)
    wd = jnp.concatenate([wu[:, W:], jnp.zeros((1, W), jnp.float32)], axis=1)
    wr = jnp.concatenate([wl[:, 1:], jnp.zeros((1, 1), jnp.float32)], axis=1)
    deg = wu + wd + wl + wr
    aff_ref[0] = jnp.concatenate([wu, wd, wl, wr, deg], axis=0)


def _cg_kernel(aff_ref, src_ref, mask_ref, lam_ref, out_ref):
    lam = lam_ref[0, 0]
    wu = aff_ref[:, 0]
    wd = aff_ref[:, 1]
    wl = aff_ref[:, 2]
    wr = aff_ref[:, 3]
    deg = aff_ref[:, 4]
    E = (jax.lax.broadcasted_iota(jnp.int32, (H, HC), 0) // S
         == jax.lax.broadcasted_iota(jnp.int32, (H, HC), 1)).astype(jnp.float32)
    Et = (jax.lax.broadcasted_iota(jnp.int32, (HC, H), 1) // S
          == jax.lax.broadcasted_iota(jnp.int32, (HC, H), 0)).astype(jnp.float32)
    inv = 1.0 / float(S * S)

    zrow = jnp.zeros((B, 1, W), jnp.float32)
    zcol = jnp.zeros((B, H, 1), jnp.float32)

    def A_op(y):
        nu = jnp.concatenate([zrow, y[:, : H - 1, :]], axis=1)
        nd = jnp.concatenate([y[:, 1:, :], zrow], axis=1)
        nl = jnp.concatenate([zcol, y[:, :, : W - 1]], axis=2)
        nr = jnp.concatenate([y[:, :, 1:], zcol], axis=2)
        Ly = deg * y - (wu * nu + wd * nd + wl * nl + wr * nr)
        ups = []
        for k in range(B):
            dk = jax.lax.dot(jax.lax.dot(Et, y[k]), E) * inv
            zk = mask_ref[k] * dk
            ups.append((jax.lax.dot(jax.lax.dot(E, zk), Et) * inv).reshape(1, H, W))
        return Ly + lam * jnp.concatenate(ups, axis=0)

    bs = []
    x0s = []
    for k in range(B):
        ms = mask_ref[k] * src_ref[k]
        bs.append((jax.lax.dot(jax.lax.dot(E, ms), Et) * inv).reshape(1, H, W))
        x0s.append(jax.lax.dot(jax.lax.dot(E, src_ref[k]), Et).reshape(1, H, W))
    b = lam * jnp.concatenate(bs, axis=0)
    x = jnp.concatenate(x0s, axis=0)

    r = b - A_op(x)
    p = r
    rs = jnp.sum(r * r)

    def body(_, carry):
        x, r, p, rs = carry
        Ap = A_op(p)
        alpha = rs / (jnp.sum(p * Ap) + 1e-12)
        x = x + alpha * p
        r = r - alpha * Ap
        rs_new = jnp.sum(r * r)
        p = r + (rs_new / (rs + 1e-12)) * p
        return x, r, p, rs_new

    x, r, p, rs = jax.lax.fori_loop(0, NIT, body, (x, r, p, rs))
    out_ref[...] = x


def kernel(guide, source, mask_lr, y_bicubic, var_w, var_b, fe_w1, fe_b1,
           fe_w2, fe_b2, fe_w3, fe_b3, log_lambda, log_mu):
    mu = jnp.exp(log_mu).reshape(1, 1)
    lam = jnp.exp(log_lambda).reshape(1, 1)

    g_f = guide.reshape(B, 3, N)
    yb_f = y_bicubic.reshape(B, 1, N)

    w1c = jnp.concatenate([fe_w1, var_w], axis=0)
    w1_flat = w1c.transpose(0, 2, 3, 1).reshape(33, 36)
    b1c = jnp.concatenate([fe_b1, var_b], axis=0).reshape(33, 1)
    w2r = fe_w2.transpose(0, 2, 3, 1).reshape(32, 288)
    w3r = fe_w3.transpose(0, 2, 3, 1).reshape(32, 288)
    b2 = fe_b2.reshape(32, 1)
    b3 = fe_b3.reshape(32, 1)

    var_f, aff_f = pl.pallas_call(
        _conv_kernel,
        grid=(B,),
        in_specs=[
            pl.BlockSpec((1, 3, N), lambda b: (b, 0, 0)),
            pl.BlockSpec((1, 1, N), lambda b: (b, 0, 0)),
            pl.BlockSpec((33, 36), lambda b: (0, 0)),
            pl.BlockSpec((33, 1), lambda b: (0, 0)),
            pl.BlockSpec((32, 288), lambda b: (0, 0)),
            pl.BlockSpec((32, 1), lambda b: (0, 0)),
            pl.BlockSpec((32, 288), lambda b: (0, 0)),
            pl.BlockSpec((32, 1), lambda b: (0, 0)),
            pl.BlockSpec((1, 1), lambda b: (0, 0)),
        ],
        out_specs=[
            pl.BlockSpec((1, 1, N), lambda b: (b, 0, 0)),
            pl.BlockSpec((1, 5, N), lambda b: (b, 0, 0)),
        ],
        out_shape=[
            jax.ShapeDtypeStruct((B, 1, N), jnp.float32),
            jax.ShapeDtypeStruct((B, 5, N), jnp.float32),
        ],
        compiler_params=pltpu.CompilerParams(
            dimension_semantics=("parallel",)),
    )(g_f, yb_f, w1_flat, b1c, w2r, b2, w3r, b3, mu)

    var = var_f.reshape(B, 1, H, W)
    aff = aff_f.reshape(B, 5, H, W)

    src = source.reshape(B, HC, HC)
    msk = mask_lr.reshape(B, HC, HC)
    aff3 = aff_f.reshape(B, 5, H, W)

    y = pl.pallas_call(
        _cg_kernel,
        grid=(1,),
        in_specs=[
            pl.BlockSpec((B, 5, H, W), lambda i: (0, 0, 0, 0)),
            pl.BlockSpec((B, HC, HC), lambda i: (0, 0, 0)),
            pl.BlockSpec((B, HC, HC), lambda i: (0, 0, 0)),
            pl.BlockSpec((1, 1), lambda i: (0, 0)),
        ],
        out_specs=pl.BlockSpec((B, H, W), lambda i: (0, 0, 0)),
        out_shape=jax.ShapeDtypeStruct((B, H, W), jnp.float32),
    )(aff3, src, msk, lam)

    return (y.reshape(B, 1, H, W), var, aff)


# speedup vs baseline: 1.2389x; 1.0006x over previous
import jax
import jax.numpy as jnp
from jax.experimental import pallas as pl
from jax.experimental.pallas import tpu as pltpu

H = 256
W = 256
N = H * W
S = 8
HC = H // S
B = 4
NIT = 30


def _shift_flat(x, dh, dw, col_ids):
    s = W * dh + dw
    C = x.shape[0]
    if s > 0:
        x = jnp.concatenate([jnp.zeros((C, s), jnp.float32), x[:, : N - s]], axis=1)
    elif s < 0:
        x = jnp.concatenate([x[:, -s:], jnp.zeros((C, -s), jnp.float32)], axis=1)
    if dw == 1:
        x = jnp.where(col_ids == 0, 0.0, x)
    elif dw == -1:
        x = jnp.where(col_ids == W - 1, 0.0, x)
    return x


def _conv_kernel(g_ref, yb_ref, w1_ref, b1_ref, w2_ref, b2_ref, w3_ref, b3_ref,
                 mu_ref, var_ref, aff_ref):
    col_ids = jax.lax.broadcasted_iota(jnp.int32, (1, N), 1) % W
    x0 = jnp.concatenate([g_ref[0], yb_ref[0]], axis=0)

    cols = [_shift_flat(x0, 1 - i, 1 - j, col_ids) for i in range(3) for j in range(3)]
    im2col = jnp.concatenate(cols, axis=0)
    l1 = jax.lax.dot(w1_ref[...], im2col, preferred_element_type=jnp.float32)
    l1 = l1 + b1_ref[...]
    var_ref[0] = l1[32:33]
    f = jnp.maximum(l1[:32], 0.0)

    CH = 4096
    PAD = 512
    for w_ref, b_ref, relu in ((w2_ref, b2_ref, True), (w3_ref, b3_ref, False)):
        fpad = jnp.concatenate(
            [jnp.zeros((32, PAD), jnp.float32), f,
             jnp.zeros((32, PAD), jnp.float32)], axis=1)
        outs = []
        for c in range(N // CH):
            base = c * CH
            cols = []
            for i in range(3):
                for j in range(3):
                    s = W * (1 - i) + (1 - j)
                    st = base + PAD - s
                    sl = fpad[:, st:st + CH]
                    cc = col_ids[:, :CH]
                    if j == 0:
                        sl = jnp.where(cc == 0, 0.0, sl)
                    elif j == 2:
                        sl = jnp.where(cc == W - 1, 0.0, sl)
                    cols.append(sl)
            im2col_c = jnp.concatenate(cols, axis=0)
            outs.append(jax.lax.dot(w_ref[...], im2col_c,
                                    preferred_element_type=jnp.float32))
        f = jnp.concatenate(outs, axis=1) + b_ref[...]
        if relu:
            f = jnp.maximum(f, 0.0)

    mu = mu_ref[0, 0]
    row_ids = jax.lax.broadcasted_iota(jnp.int32, (1, N), 1) // W
    s = jnp.sum(f * f, axis=0, keepdims=True)

    def aff(dh, dw, border_ids, border_val):
        fn = _shift_flat(f, dh, dw, col_ids)
        cross = jnp.sum(f * fn, axis=0, keepdims=True)
        d2 = s + _shift_flat(s, dh, dw, col_ids) - 2.0 * cross
        wdir = jnp.exp(-d2 / mu)
        return jnp.where(border_ids == border_val, 0.0, wdir)

    wu = aff(1, 0, row_ids, 0)
    wl = aff(0, 1, col_ids, 0)
    wd = jnp.concatenate([wu[:, W:], jnp.zeros((1, W), jnp.float32)], axis=1)
    wr = jnp.concatenate([wl[:, 1:], jnp.zeros((1, 1), jnp.float32)], axis=1)
    deg = wu + wd + wl + wr
    aff_ref[0] = jnp.concatenate([wu, wd, wl, wr, deg], axis=0)


def _cg_kernel(aff_ref, src_ref, mask_ref, lam_ref, out_ref):
    lam = lam_ref[0, 0]
    wu = aff_ref[:, 0]
    wd = aff_ref[:, 1]
    wl = aff_ref[:, 2]
    wr = aff_ref[:, 3]
    deg = aff_ref[:, 4]
    E = (jax.lax.broadcasted_iota(jnp.int32, (H, HC), 0) // S
         == jax.lax.broadcasted_iota(jnp.int32, (H, HC), 1)).astype(jnp.float32)
    Et = (jax.lax.broadcasted_iota(jnp.int32, (HC, H), 1) // S
          == jax.lax.broadcasted_iota(jnp.int32, (HC, H), 0)).astype(jnp.float32)
    inv = 1.0 / float(S * S)

    zrow = jnp.zeros((B, 1, W), jnp.float32)
    zcol = jnp.zeros((B, H, 1), jnp.float32)

    def A_op(y):
        nu = jnp.concatenate([zrow, y[:, : H - 1, :]], axis=1)
        nd = jnp.concatenate([y[:, 1:, :], zrow], axis=1)
        nl = jnp.concatenate([zcol, y[:, :, : W - 1]], axis=2)
        nr = jnp.concatenate([y[:, :, 1:], zcol], axis=2)
        Ly = deg * y - (wu * nu + wd * nd + wl * nl + wr * nr)
        ups = []
        for k in range(B):
            dk = jax.lax.dot(jax.lax.dot(Et, y[k]), E) * inv
            zk = mask_ref[k] * dk
            ups.append((jax.lax.dot(jax.lax.dot(E, zk), Et) * inv).reshape(1, H, W))
        return Ly + lam * jnp.concatenate(ups, axis=0)

    bs = []
    x0s = []
    for k in range(B):
        ms = mask_ref[k] * src_ref[k]
        bs.append((jax.lax.dot(jax.lax.dot(E, ms), Et) * inv).reshape(1, H, W))
        x0s.append(jax.lax.dot(jax.lax.dot(E, src_ref[k]), Et).reshape(1, H, W))
    b = lam * jnp.concatenate(bs, axis=0)
    x = jnp.concatenate(x0s, axis=0)

    r = b - A_op(x)
    p = r
    rs = jnp.sum(r * r)

    def body(_, carry):
        x, r, p, rs = carry
        Ap = A_op(p)
        alpha = rs / (jnp.sum(p * Ap) + 1e-12)
        x = x + alpha * p
        r = r - alpha * Ap
        rs_new = jnp.sum(r * r)
        p = r + (rs_new / (rs + 1e-12)) * p
        return x, r, p, rs_new

    x, r, p, rs = jax.lax.fori_loop(0, NIT, body, (x, r, p, rs))
    out_ref[...] = x


def kernel(guide, source, mask_lr, y_bicubic, var_w, var_b, fe_w1, fe_b1,
           fe_w2, fe_b2, fe_w3, fe_b3, log_lambda, log_mu):
    mu = jnp.exp(log_mu).reshape(1, 1)
    lam = jnp.exp(log_lambda).reshape(1, 1)

    g_f = guide.reshape(B, 3, N)
    yb_f = y_bicubic.reshape(B, 1, N)

    w1c = jnp.concatenate([fe_w1, var_w], axis=0)
    w1_flat = w1c.transpose(0, 2, 3, 1).reshape(33, 36)
    b1c = jnp.concatenate([fe_b1, var_b], axis=0).reshape(33, 1)
    w2r = fe_w2.transpose(0, 2, 3, 1).reshape(32, 288)
    w3r = fe_w3.transpose(0, 2, 3, 1).reshape(32, 288)
    b2 = fe_b2.reshape(32, 1)
    b3 = fe_b3.reshape(32, 1)

    var_f, aff_f = pl.pallas_call(
        _conv_kernel,
        grid=(B,),
        in_specs=[
            pl.BlockSpec((1, 3, N), lambda b: (b, 0, 0)),
            pl.BlockSpec((1, 1, N), lambda b: (b, 0, 0)),
            pl.BlockSpec((33, 36), lambda b: (0, 0)),
            pl.BlockSpec((33, 1), lambda b: (0, 0)),
            pl.BlockSpec((32, 288), lambda b: (0, 0)),
            pl.BlockSpec((32, 1), lambda b: (0, 0)),
            pl.BlockSpec((32, 288), lambda b: (0, 0)),
            pl.BlockSpec((32, 1), lambda b: (0, 0)),
            pl.BlockSpec((1, 1), lambda b: (0, 0)),
        ],
        out_specs=[
            pl.BlockSpec((1, 1, N), lambda b: (b, 0, 0)),
            pl.BlockSpec((1, 5, N), lambda b: (b, 0, 0)),
        ],
        out_shape=[
            jax.ShapeDtypeStruct((B, 1, N), jnp.float32),
            jax.ShapeDtypeStruct((B, 5, N), jnp.float32),
        ],
        compiler_params=pltpu.CompilerParams(
            dimension_semantics=("parallel",)),
    )(g_f, yb_f, w1_flat, b1c, w2r, b2, w3r, b3, mu)

    var = var_f.reshape(B, 1, H, W)
    aff = aff_f.reshape(B, 5, H, W)

    src = source.reshape(B, HC, HC)
    msk = mask_lr.reshape(B, HC, HC)
    aff3 = aff_f.reshape(B, 5, H, W)

    y = pl.pallas_call(
        _cg_kernel,
        grid=(1,),
        in_specs=[
            pl.BlockSpec((B, 5, H, W), lambda i: (0, 0, 0, 0)),
            pl.BlockSpec((B, HC, HC), lambda i: (0, 0, 0)),
            pl.BlockSpec((B, HC, HC), lambda i: (0, 0, 0)),
            pl.BlockSpec((1, 1), lambda i: (0, 0)),
        ],
        out_specs=pl.BlockSpec((B, H, W), lambda i: (0, 0, 0)),
        out_shape=jax.ShapeDtypeStruct((B, H, W), jnp.float32),
    )(aff3, src, msk, lam)

    return (y.reshape(B, 1, H, W), var, aff)


# speedup vs baseline: 1.3517x; 1.0911x over previous
import jax
import jax.numpy as jnp
from jax.experimental import pallas as pl
from jax.experimental.pallas import tpu as pltpu

H = 256
W = 256
N = H * W
S = 8
HC = H // S
B = 4
NIT = 30


def _shift_flat(x, dh, dw, col_ids):
    s = W * dh + dw
    C = x.shape[0]
    if s > 0:
        x = jnp.concatenate([jnp.zeros((C, s), jnp.float32), x[:, : N - s]], axis=1)
    elif s < 0:
        x = jnp.concatenate([x[:, -s:], jnp.zeros((C, -s), jnp.float32)], axis=1)
    if dw == 1:
        x = jnp.where(col_ids == 0, 0.0, x)
    elif dw == -1:
        x = jnp.where(col_ids == W - 1, 0.0, x)
    return x


def _conv_kernel(g_ref, yb_ref, w1_ref, b1_ref, w2_ref, b2_ref, w3_ref, b3_ref,
                 mu_ref, var_ref, aff_ref):
    col_ids = jax.lax.broadcasted_iota(jnp.int32, (1, N), 1) % W
    x0 = jnp.concatenate([g_ref[0], yb_ref[0]], axis=0)

    cols = [_shift_flat(x0, 1 - i, 1 - j, col_ids) for i in range(3) for j in range(3)]
    im2col = jnp.concatenate(cols, axis=0)
    l1 = jax.lax.dot(w1_ref[...], im2col, preferred_element_type=jnp.float32)
    l1 = l1 + b1_ref[...]
    var_ref[0] = l1[32:33]
    f = jnp.maximum(l1[:32], 0.0)

    CH = 4096
    PAD = 512
    for w_ref, b_ref, relu in ((w2_ref, b2_ref, True), (w3_ref, b3_ref, False)):
        fpad = jnp.concatenate(
            [jnp.zeros((32, PAD), jnp.float32), f,
             jnp.zeros((32, PAD), jnp.float32)], axis=1)
        outs = []
        for c in range(N // CH):
            base = c * CH
            cols = []
            for i in range(3):
                for j in range(3):
                    s = W * (1 - i) + (1 - j)
                    st = base + PAD - s
                    sl = fpad[:, st:st + CH]
                    cc = col_ids[:, :CH]
                    if j == 0:
                        sl = jnp.where(cc == 0, 0.0, sl)
                    elif j == 2:
                        sl = jnp.where(cc == W - 1, 0.0, sl)
                    cols.append(sl)
            im2col_c = jnp.concatenate(cols, axis=0)
            outs.append(jax.lax.dot(w_ref[...], im2col_c,
                                    preferred_element_type=jnp.float32))
        f = jnp.concatenate(outs, axis=1) + b_ref[...]
        if relu:
            f = jnp.maximum(f, 0.0)

    mu = mu_ref[0, 0]
    row_ids = jax.lax.broadcasted_iota(jnp.int32, (1, N), 1) // W
    s = jnp.sum(f * f, axis=0, keepdims=True)

    def aff(dh, dw, border_ids, border_val):
        fn = _shift_flat(f, dh, dw, col_ids)
        cross = jnp.sum(f * fn, axis=0, keepdims=True)
        d2 = s + _shift_flat(s, dh, dw, col_ids) - 2.0 * cross
        wdir = jnp.exp(-d2 / mu)
        return jnp.where(border_ids == border_val, 0.0, wdir)

    wu = aff(1, 0, row_ids, 0)
    wl = aff(0, 1, col_ids, 0)
    wd = jnp.concatenate([wu[:, W:], jnp.zeros((1, W), jnp.float32)], axis=1)
    wr = jnp.concatenate([wl[:, 1:], jnp.zeros((1, 1), jnp.float32)], axis=1)
    deg = wu + wd + wl + wr
    aff_ref[0] = jnp.concatenate([wu, wd, wl, wr, deg], axis=0)


def _cg_kernel(aff_ref, src_ref, mask_ref, lam_ref, out_ref):
    lam = lam_ref[0, 0]
    wu = aff_ref[:, 0]
    wd = aff_ref[:, 1]
    wl = aff_ref[:, 2]
    wr = aff_ref[:, 3]
    deg = aff_ref[:, 4]
    E = (jax.lax.broadcasted_iota(jnp.int32, (H, HC), 0) // S
         == jax.lax.broadcasted_iota(jnp.int32, (H, HC), 1)).astype(jnp.float32)
    Et = (jax.lax.broadcasted_iota(jnp.int32, (HC, H), 1) // S
          == jax.lax.broadcasted_iota(jnp.int32, (HC, H), 0)).astype(jnp.float32)
    inv = 1.0 / float(S * S)

    zrow = jnp.zeros((B, 1, W), jnp.float32)
    zcol = jnp.zeros((B, H, 1), jnp.float32)

    def A_op(y):
        nu = jnp.concatenate([zrow, y[:, : H - 1, :]], axis=1)
        nd = jnp.concatenate([y[:, 1:, :], zrow], axis=1)
        nl = jnp.concatenate([zcol, y[:, :, : W - 1]], axis=2)
        nr = jnp.concatenate([y[:, :, 1:], zcol], axis=2)
        Ly = deg * y - (wu * nu + wd * nd + wl * nl + wr * nr)
        ups = []
        for k in range(B):
            dk = jax.lax.dot(jax.lax.dot(Et, y[k]), E) * inv
            zk = mask_ref[k] * dk
            ups.append((jax.lax.dot(jax.lax.dot(E, zk), Et) * inv).reshape(1, H, W))
        return Ly + lam * jnp.concatenate(ups, axis=0)

    bs = []
    x0s = []
    for k in range(B):
        ms = mask_ref[k] * src_ref[k]
        bs.append((jax.lax.dot(jax.lax.dot(E, ms), Et) * inv).reshape(1, H, W))
        x0s.append(jax.lax.dot(jax.lax.dot(E, src_ref[k]), Et).reshape(1, H, W))
    b = lam * jnp.concatenate(bs, axis=0)
    x = jnp.concatenate(x0s, axis=0)

    r = b - A_op(x)
    p = r
    rs = jnp.sum(r * r)

    for _ in range(NIT):
        Ap = A_op(p)
        alpha = rs / (jnp.sum(p * Ap) + 1e-12)
        x = x + alpha * p
        r = r - alpha * Ap
        rs_new = jnp.sum(r * r)
        p = r + (rs_new / (rs + 1e-12)) * p
        rs = rs_new
    out_ref[...] = x


def kernel(guide, source, mask_lr, y_bicubic, var_w, var_b, fe_w1, fe_b1,
           fe_w2, fe_b2, fe_w3, fe_b3, log_lambda, log_mu):
    mu = jnp.exp(log_mu).reshape(1, 1)
    lam = jnp.exp(log_lambda).reshape(1, 1)

    g_f = guide.reshape(B, 3, N)
    yb_f = y_bicubic.reshape(B, 1, N)

    w1c = jnp.concatenate([fe_w1, var_w], axis=0)
    w1_flat = w1c.transpose(0, 2, 3, 1).reshape(33, 36)
    b1c = jnp.concatenate([fe_b1, var_b], axis=0).reshape(33, 1)
    w2r = fe_w2.transpose(0, 2, 3, 1).reshape(32, 288)
    w3r = fe_w3.transpose(0, 2, 3, 1).reshape(32, 288)
    b2 = fe_b2.reshape(32, 1)
    b3 = fe_b3.reshape(32, 1)

    var_f, aff_f = pl.pallas_call(
        _conv_kernel,
        grid=(B,),
        in_specs=[
            pl.BlockSpec((1, 3, N), lambda b: (b, 0, 0)),
            pl.BlockSpec((1, 1, N), lambda b: (b, 0, 0)),
            pl.BlockSpec((33, 36), lambda b: (0, 0)),
            pl.BlockSpec((33, 1), lambda b: (0, 0)),
            pl.BlockSpec((32, 288), lambda b: (0, 0)),
            pl.BlockSpec((32, 1), lambda b: (0, 0)),
            pl.BlockSpec((32, 288), lambda b: (0, 0)),
            pl.BlockSpec((32, 1), lambda b: (0, 0)),
            pl.BlockSpec((1, 1), lambda b: (0, 0)),
        ],
        out_specs=[
            pl.BlockSpec((1, 1, N), lambda b: (b, 0, 0)),
            pl.BlockSpec((1, 5, N), lambda b: (b, 0, 0)),
        ],
        out_shape=[
            jax.ShapeDtypeStruct((B, 1, N), jnp.float32),
            jax.ShapeDtypeStruct((B, 5, N), jnp.float32),
        ],
        compiler_params=pltpu.CompilerParams(
            dimension_semantics=("parallel",)),
    )(g_f, yb_f, w1_flat, b1c, w2r, b2, w3r, b3, mu)

    var = var_f.reshape(B, 1, H, W)
    aff = aff_f.reshape(B, 5, H, W)

    src = source.reshape(B, HC, HC)
    msk = mask_lr.reshape(B, HC, HC)
    aff3 = aff_f.reshape(B, 5, H, W)

    y = pl.pallas_call(
        _cg_kernel,
        grid=(1,),
        in_specs=[
            pl.BlockSpec((B, 5, H, W), lambda i: (0, 0, 0, 0)),
            pl.BlockSpec((B, HC, HC), lambda i: (0, 0, 0)),
            pl.BlockSpec((B, HC, HC), lambda i: (0, 0, 0)),
            pl.BlockSpec((1, 1), lambda i: (0, 0)),
        ],
        out_specs=pl.BlockSpec((B, H, W), lambda i: (0, 0, 0)),
        out_shape=jax.ShapeDtypeStruct((B, H, W), jnp.float32),
    )(aff3, src, msk, lam)

    return (y.reshape(B, 1, H, W), var, aff)
